# Initial kernel scaffold; baseline (speedup 1.0000x reference)
import jax
import jax.numpy as jnp
from jax.experimental import pallas as pl


def kernel(target_protos, ref_protos, k):
    raise NotImplementedError("write your pallas kernel here")


# speedup vs baseline: 8.6810x; 8.6810x over previous
import functools

import jax
import jax.numpy as jnp
from jax import lax
from jax.experimental import pallas as pl
from jax.experimental.pallas import tpu as pltpu
from jax.experimental.pallas import tpu_sc as plsc

B, N, M, C, K = 4, 1024, 8192, 512, 16
MC = 8
MT = M // MC
CAND = MC * K


def _topk_body(t_ref, r_ref, tss_ref, rss_ref, idx_ref, vals_s, gidx_s):
    b = pl.program_id(0)
    mc = pl.program_id(1)

    t = t_ref[0]
    r = r_ref[0]
    tn = t / (jnp.sqrt(tss_ref[0]) + 1e-8)
    rn = r / (jnp.sqrt(rss_ref[0]) + 1e-8)
    sim = lax.dot_general(tn, rn, (((1,), (1,)), ((), ())),
                          preferred_element_type=jnp.float32)

    iota = lax.broadcasted_iota(jnp.int32, (N, MT), 1)
    base = b * M + mc * MT
    vals_list, gidx_list = [], []
    for _ in range(K):
        m = jnp.max(sim, axis=1)
        ismax = sim == m[:, None]
        pos = jnp.min(jnp.where(ismax, iota, M), axis=1)
        vals_list.append(m)
        gidx_list.append(pos + base)
        sim = jnp.where(iota == pos[:, None], -jnp.inf, sim)
    vals_s[pl.ds(mc * K, K), :] = jnp.stack(vals_list, axis=0)
    gidx_s[pl.ds(mc * K, K), :] = jnp.stack(gidx_list, axis=0)

    @pl.when(mc == MC - 1)
    def _final():
        vcur = vals_s[...]
        gidx = gidx_s[...]
        citer = lax.broadcasted_iota(jnp.int32, (CAND, N), 0)
        outs = []
        for _ in range(K):
            m = jnp.max(vcur, axis=0)
            ismax = vcur == m[None, :]
            pos = jnp.min(jnp.where(ismax, citer, CAND), axis=0)
            sel = citer == pos[None, :]
            outs.append(jnp.sum(jnp.where(sel, gidx, 0), axis=0))
            vcur = jnp.where(sel, -jnp.inf, vcur)
        idx_ref[0] = jnp.stack(outs, axis=1)


_topk = pl.pallas_call(
    _topk_body,
    grid=(B, MC),
    in_specs=[
        pl.BlockSpec((1, N, C), lambda b, mc: (b, 0, 0)),
        pl.BlockSpec((1, MT, C), lambda b, mc: (b, mc, 0)),
        pl.BlockSpec((1, N, 1), lambda b, mc: (b, 0, 0)),
        pl.BlockSpec((1, MT, 1), lambda b, mc: (b, mc, 0)),
    ],
    out_specs=pl.BlockSpec((1, N, K), lambda b, mc: (b, 0, 0)),
    out_shape=jax.ShapeDtypeStruct((B, N, K), jnp.int32),
    scratch_shapes=[
        pltpu.VMEM((CAND, N), jnp.float32),
        pltpu.VMEM((CAND, N), jnp.int32),
    ],
)


NW = 32
RW = (B * N * K) // NW
CH = 128
NI = RW // CH


def _gather_body(tab_ref, idx_ref, out_ref, idx_c, rows_v, sem):
    c = lax.axis_index("c")
    s = lax.axis_index("s")
    wid = s * 2 + c
    base = wid * RW

    def step(i, carry):
        off = base + i * CH
        pltpu.sync_copy(idx_ref.at[pl.ds(off, CH)], idx_c)
        pltpu.async_copy(tab_ref.at[idx_c], rows_v, sem).wait()
        pltpu.sync_copy(rows_v, out_ref.at[pl.ds(off, CH)])
        return carry

    lax.fori_loop(0, NI, step, 0)


_gather = pl.kernel(
    _gather_body,
    out_type=jax.ShapeDtypeStruct((B * N * K, C), jnp.float32),
    mesh=plsc.VectorSubcoreMesh(core_axis_name="c", subcore_axis_name="s"),
    scratch_types=[
        pltpu.VMEM((CH,), jnp.int32),
        pltpu.VMEM((CH, C), jnp.float32),
        pltpu.SemaphoreType.DMA,
    ],
)


CT = 512


def _tr_body(g_ref, o_ref):
    o_ref[0] = jnp.swapaxes(g_ref[0], 0, 1)


_transpose = pl.pallas_call(
    _tr_body,
    grid=(B, (K * C) // CT),
    in_specs=[pl.BlockSpec((1, N, CT), lambda b, t: (b, 0, t))],
    out_specs=pl.BlockSpec((1, CT, N), lambda b, t: (b, t, 0)),
    out_shape=jax.ShapeDtypeStruct((B, K * C, N), jnp.float32),
)


def kernel(target_protos, ref_protos, k):
    del k
    tss = jnp.sum(target_protos * target_protos, axis=2, keepdims=True)
    rss = jnp.sum(ref_protos * ref_protos, axis=2, keepdims=True)
    idx = _topk(target_protos, ref_protos, tss, rss)
    ref_flat = ref_protos.reshape(B * M, C)
    gathered = _gather(ref_flat, idx.reshape(-1))
    out = _transpose(gathered.reshape(B, N, K * C))
    return out.reshape(B, K, C, N)
